# Initial kernel scaffold; baseline (speedup 1.0000x reference)
import jax
import jax.numpy as jnp
from jax.experimental import pallas as pl


def kernel(x, edge_index, W1l, b1, W1r, W2l, b2, W2r):
    raise NotImplementedError("write your pallas kernel here")


# speedup vs baseline: 4.6369x; 4.6369x over previous
import functools

import jax
import jax.numpy as jnp
from jax import lax
from jax.experimental import pallas as pl
from jax.experimental.pallas import tpu as pltpu
from jax.experimental.pallas import tpu_sc as plsc

N = 10000
D = 128
E = 320000
CHUNK = 128
NW = 32
CPW = (E + NW * CHUNK - 1) // (NW * CHUNK)
EPAD = NW * CPW * CHUNK
SEGN = 10240
OPS = SEGN // 16
TPAD = 16
DW = 16
ZB = 16


def _seg_body(x_hbm, src_hbm, dst_hbm, acc_out, sidx_v, didx_v, rows_v,
              zbuf, acc_s, sem):
  c = lax.axis_index("c")
  s = lax.axis_index("s")
  w = c * 16 + s

  def fill(i, _):
    zero16 = jnp.zeros((16,), jnp.float32)
    for k in range(D // 16):
      zbuf[i, pl.ds(k * 16, 16)] = zero16
    return 0
  lax.fori_loop(0, ZB, fill, 0)

  def zero_blk(t, _):
    pltpu.sync_copy(zbuf, acc_s.at[pl.ds(s * OPS + t * ZB, ZB)])
    return 0
  lax.fori_loop(0, OPS // ZB, zero_blk, 0)
  plsc.subcore_barrier()

  def chunk_body(k, _):
    base = (w * CPW + k) * CHUNK
    pltpu.sync_copy(src_hbm.at[pl.ds(base, CHUNK)], sidx_v)
    pltpu.sync_copy(dst_hbm.at[pl.ds(base, CHUNK)], didx_v)
    pltpu.async_copy(x_hbm.at[sidx_v], rows_v, sem).wait()
    pltpu.sync_copy(rows_v, acc_s.at[didx_v], add=True)
    return 0
  lax.fori_loop(0, CPW, chunk_body, 0)
  plsc.subcore_barrier()

  pltpu.sync_copy(acc_s.at[pl.ds(s * OPS, OPS)],
                  acc_out.at[pl.ds(c * SEGN + s * OPS, OPS)])


def _make_seg():
  mesh = plsc.VectorSubcoreMesh(core_axis_name="c", subcore_axis_name="s")
  scratch = [
      pltpu.VMEM((CHUNK,), jnp.int32),
      pltpu.VMEM((CHUNK,), jnp.int32),
      pltpu.VMEM((CHUNK, D), jnp.float32),
      pltpu.VMEM((ZB, D), jnp.float32),
      pltpu.VMEM_SHARED((SEGN, D), jnp.float32),
      pltpu.SemaphoreType.DMA,
  ]
  return pl.kernel(
      _seg_body,
      out_type=jax.ShapeDtypeStruct((2 * SEGN, D), jnp.float32),
      mesh=mesh,
      scratch_types=scratch,
  )


def _layer_body(relu, a0, a1, d0, d1, xb, wl, bb, wr, ob):
  deg = d0[...][:, :1] + d1[...][:, :1]
  mean = (a0[...] + a1[...]) / jnp.maximum(deg, 1.0)
  z = (jnp.dot(mean, wl[...], preferred_element_type=jnp.float32,
               precision=lax.Precision.HIGHEST)
       + jnp.dot(xb[...], wr[...], preferred_element_type=jnp.float32,
                 precision=lax.Precision.HIGHEST)
       + bb[...])
  ob[...] = jnp.maximum(z, 0.0) if relu else z


def _layer(acc, deg, x, Wl, b, Wr, relu):
  BN = 1000
  grid = (N // BN,)
  row_spec = pl.BlockSpec((BN, D), lambda i: (i, 0))
  deg_spec = pl.BlockSpec((BN, D), lambda i: (i, 0))
  w_spec = pl.BlockSpec((D, D), lambda i: (0, 0))
  b_spec = pl.BlockSpec((1, D), lambda i: (0, 0))
  return pl.pallas_call(
      functools.partial(_layer_body, relu),
      grid=grid,
      in_specs=[row_spec, row_spec, deg_spec, deg_spec, row_spec,
                w_spec, b_spec, w_spec],
      out_specs=row_spec,
      out_shape=jax.ShapeDtypeStruct((N, D), jnp.float32),
  )(acc[0], acc[1], deg[0], deg[1], x, Wl, b.reshape(1, D), Wr)


_seg = _make_seg()


def kernel(x, edge_index, W1l, b1, W1r, W2l, b2, W2r):
  src = edge_index[0].astype(jnp.int32)
  dst = edge_index[1].astype(jnp.int32)
  pad = EPAD - E
  pad_src = N + (jnp.arange(pad, dtype=jnp.int32) % TPAD)
  pad_dst = N + (jnp.arange(pad, dtype=jnp.int32) % (SEGN - N))
  srcp = jnp.concatenate([src, pad_src])
  dstp = jnp.concatenate([dst, pad_dst])

  xp = jnp.concatenate([x, jnp.zeros((TPAD, D), jnp.float32)])
  ones_tab = jnp.concatenate(
      [jnp.ones((N, D), jnp.float32), jnp.zeros((TPAD, D), jnp.float32)])
  a1f = _seg(xp, srcp, dstp)
  degf = _seg(ones_tab, srcp, dstp)
  acc1 = (a1f[:SEGN], a1f[SEGN:])
  deg = (degf[:SEGN], degf[SEGN:])
  h = _layer(acc1, deg, x, W1l, b1, W1r, relu=True)
  hp = jnp.concatenate([h, jnp.zeros((TPAD, D), jnp.float32)])
  a2f = _seg(hp, srcp, dstp)
  acc2 = (a2f[:SEGN], a2f[SEGN:])
  out = _layer(acc2, deg, h, W2l, b2, W2r, relu=False)
  return out


# speedup vs baseline: 5.5950x; 1.2066x over previous
import functools

import jax
import jax.numpy as jnp
from jax import lax
from jax.experimental import pallas as pl
from jax.experimental.pallas import tpu as pltpu
from jax.experimental.pallas import tpu_sc as plsc

N = 10000
D = 128
E = 320000
CHUNK = 128
NW = 32
CPW = (E + NW * CHUNK - 1) // (NW * CHUNK)
EPAD = NW * CPW * CHUNK
SEGN = 10240
OPS = SEGN // 16
TPAD = 16
DW = 16
ZB = 16


def _seg_body(x_hbm, src_hbm, dst_hbm, acc_out, sidx_v, didx_v, rows_v,
              zbuf, acc_s, sem):
  c = lax.axis_index("c")
  s = lax.axis_index("s")
  w = c * 16 + s

  def fill(i, _):
    zero16 = jnp.zeros((16,), jnp.float32)
    for k in range(D // 16):
      zbuf[i, pl.ds(k * 16, 16)] = zero16
    return 0
  lax.fori_loop(0, ZB, fill, 0)

  def zero_blk(t, _):
    pltpu.sync_copy(zbuf, acc_s.at[pl.ds(s * OPS + t * ZB, ZB)])
    return 0
  lax.fori_loop(0, OPS // ZB, zero_blk, 0)
  plsc.subcore_barrier()

  def chunk_body(k, _):
    base = (w * CPW + k) * CHUNK
    pltpu.sync_copy(src_hbm.at[pl.ds(base, CHUNK)], sidx_v)
    pltpu.sync_copy(dst_hbm.at[pl.ds(base, CHUNK)], didx_v)
    pltpu.async_copy(x_hbm.at[sidx_v], rows_v, sem).wait()
    pltpu.sync_copy(rows_v, acc_s.at[didx_v], add=True)
    return 0
  lax.fori_loop(0, CPW, chunk_body, 0)
  plsc.subcore_barrier()

  pltpu.sync_copy(acc_s.at[pl.ds(s * OPS, OPS)],
                  acc_out.at[pl.ds(c * SEGN + s * OPS, OPS)])


def _deg_body(dst_hbm, deg_out, didx_v, ones_v, zbuf, acc_s):
  c = lax.axis_index("c")
  s = lax.axis_index("s")
  w = c * 16 + s

  def fill(i, _):
    for k in range(D // 16):
      zbuf[i, pl.ds(k * 16, 16)] = jnp.zeros((16,), jnp.float32)
    return 0
  lax.fori_loop(0, ZB, fill, 0)

  def fill2(i, _):
    for k in range(D // 16):
      ones_v[i, pl.ds(k * 16, 16)] = jnp.ones((16,), jnp.float32)
    return 0
  lax.fori_loop(0, CHUNK, fill2, 0)

  def zero_blk(t, _):
    pltpu.sync_copy(zbuf, acc_s.at[pl.ds(s * OPS + t * ZB, ZB)])
    return 0
  lax.fori_loop(0, OPS // ZB, zero_blk, 0)
  plsc.subcore_barrier()

  def chunk_body(k, _):
    base = (w * CPW + k) * CHUNK
    pltpu.sync_copy(dst_hbm.at[pl.ds(base, CHUNK)], didx_v)
    pltpu.sync_copy(ones_v, acc_s.at[didx_v], add=True)
    return 0
  lax.fori_loop(0, CPW, chunk_body, 0)
  plsc.subcore_barrier()

  pltpu.sync_copy(acc_s.at[pl.ds(s * OPS, OPS)],
                  deg_out.at[pl.ds(c * SEGN + s * OPS, OPS)])


def _make_deg():
  mesh = plsc.VectorSubcoreMesh(core_axis_name="c", subcore_axis_name="s")
  scratch = [
      pltpu.VMEM((CHUNK,), jnp.int32),
      pltpu.VMEM((CHUNK, D), jnp.float32),
      pltpu.VMEM((ZB, D), jnp.float32),
      pltpu.VMEM_SHARED((SEGN, D), jnp.float32),
  ]
  return pl.kernel(
      _deg_body,
      out_type=jax.ShapeDtypeStruct((2 * SEGN, D), jnp.float32),
      mesh=mesh,
      scratch_types=scratch,
  )


def _make_seg():
  mesh = plsc.VectorSubcoreMesh(core_axis_name="c", subcore_axis_name="s")
  scratch = [
      pltpu.VMEM((CHUNK,), jnp.int32),
      pltpu.VMEM((CHUNK,), jnp.int32),
      pltpu.VMEM((CHUNK, D), jnp.float32),
      pltpu.VMEM((ZB, D), jnp.float32),
      pltpu.VMEM_SHARED((SEGN, D), jnp.float32),
      pltpu.SemaphoreType.DMA,
  ]
  return pl.kernel(
      _seg_body,
      out_type=jax.ShapeDtypeStruct((2 * SEGN, D), jnp.float32),
      mesh=mesh,
      scratch_types=scratch,
  )


def _layer_body(relu, a0, a1, d0, d1, xb, wl, bb, wr, ob):
  deg = d0[...][:, :1] + d1[...][:, :1]
  mean = (a0[...] + a1[...]) / jnp.maximum(deg, 1.0)
  z = (jnp.dot(mean, wl[...], preferred_element_type=jnp.float32,
               precision=lax.Precision.HIGHEST)
       + jnp.dot(xb[...], wr[...], preferred_element_type=jnp.float32,
                 precision=lax.Precision.HIGHEST)
       + bb[...])
  ob[...] = jnp.maximum(z, 0.0) if relu else z


def _layer(acc, deg, x, Wl, b, Wr, relu):
  BN = 1000
  grid = (N // BN,)
  row_spec = pl.BlockSpec((BN, D), lambda i: (i, 0))
  deg_spec = pl.BlockSpec((BN, D), lambda i: (i, 0))
  w_spec = pl.BlockSpec((D, D), lambda i: (0, 0))
  b_spec = pl.BlockSpec((1, D), lambda i: (0, 0))
  return pl.pallas_call(
      functools.partial(_layer_body, relu),
      grid=grid,
      in_specs=[row_spec, row_spec, deg_spec, deg_spec, row_spec,
                w_spec, b_spec, w_spec],
      out_specs=row_spec,
      out_shape=jax.ShapeDtypeStruct((N, D), jnp.float32),
  )(acc[0], acc[1], deg[0], deg[1], x, Wl, b.reshape(1, D), Wr)


_seg = _make_seg()
_deg = _make_deg()


def kernel(x, edge_index, W1l, b1, W1r, W2l, b2, W2r):
  src = edge_index[0].astype(jnp.int32)
  dst = edge_index[1].astype(jnp.int32)
  pad = EPAD - E
  pad_src = N + (jnp.arange(pad, dtype=jnp.int32) % TPAD)
  pad_dst = N + (jnp.arange(pad, dtype=jnp.int32) % (SEGN - N))
  srcp = jnp.concatenate([src, pad_src])
  dstp = jnp.concatenate([dst, pad_dst])

  xp = jnp.concatenate([x, jnp.zeros((TPAD, D), jnp.float32)])
  a1f = _seg(xp, srcp, dstp)
  degf = _deg(dstp)
  acc1 = (a1f[:SEGN], a1f[SEGN:])
  deg = (degf[:SEGN], degf[SEGN:])
  h = _layer(acc1, deg, x, W1l, b1, W1r, relu=True)
  hp = jnp.concatenate([h, jnp.zeros((TPAD, D), jnp.float32)])
  a2f = _seg(hp, srcp, dstp)
  acc2 = (a2f[:SEGN], a2f[SEGN:])
  out = _layer(acc2, deg, h, W2l, b2, W2r, relu=False)
  return out
